# baseline (speedup 1.0000x reference)
import functools

import jax
import jax.numpy as jnp
from jax import lax
from jax.experimental import pallas as pl
from jax.experimental.pallas import tpu as pltpu
from jax.experimental.pallas import tpu_sc as plsc

B = 16384
K = 64
NC = 2
NS = 16
NW = NC * NS
BPW = B // NW
CHUNK = 128
NCHUNK = BPW // CHUNK
GROUPS = BPW // 16

_mesh = plsc.VectorSubcoreMesh(core_axis_name="c", subcore_axis_name="s")

_GATHER_DNUMS = lax.GatherDimensionNumbers(
    offset_dims=(), collapsed_slice_dims=(0,), start_index_map=(0,))

TBLK = 4096


def _tp_body(x_ref, o_ref):
    i = lax.broadcasted_iota(jnp.int32, (K, K), 0)
    j = lax.broadcasted_iota(jnp.int32, (K, K), 1)
    eye = (i == j).astype(jnp.float32)
    o_ref[...] = lax.dot_general(
        x_ref[...], eye, (((0,), (0,)), ((), ())),
        preferred_element_type=jnp.float32,
        precision=lax.Precision.HIGHEST)


def _transpose_table(t_fm):
    nrows = t_fm.shape[1]
    grid = (nrows + TBLK - 1) // TBLK
    return pl.pallas_call(
        _tp_body,
        grid=(grid,),
        in_specs=[pl.BlockSpec((K, TBLK), lambda i: (0, i))],
        out_specs=pl.BlockSpec((TBLK, K), lambda i: (i, 0)),
        out_shape=jax.ShapeDtypeStruct((nrows, K), jnp.float32),
    )(t_fm)


def _permute(x, idx):
    return lax.gather(x, idx[:, None], _GATHER_DNUMS, (1,),
                      mode=lax.GatherScatterMode.PROMISE_IN_BOUNDS)


@functools.partial(
    pl.kernel,
    out_type=(jax.ShapeDtypeStruct((NW, BPW, K), jnp.float32),
              jax.ShapeDtypeStruct((NW, NCHUNK, CHUNK), jnp.float32)),
    mesh=_mesh,
    compiler_params=pltpu.CompilerParams(use_tc_tiling_on_sc=False),
    scratch_types=[
        pltpu.VMEM((NCHUNK, CHUNK), jnp.int32),
        pltpu.VMEM((BPW, K), jnp.float32),
        pltpu.VMEM((NCHUNK, CHUNK), jnp.float32),
        pltpu.SemaphoreType.DMA,
    ],
)
def _mf_user(user_hbm, ue_hbm, ub_hbm, rows_hbm, ubias_hbm,
             idx_u, u_rows, bias_u, sem):
    wid = lax.axis_index("s") * NC + lax.axis_index("c")

    pltpu.sync_copy(user_hbm.at[wid], idx_u)
    copies = []
    for c in range(NCHUNK):
        copies.append(pltpu.async_copy(
            ue_hbm.at[idx_u.at[c]], u_rows.at[pl.ds(c * CHUNK, CHUNK)], sem))
        copies.append(pltpu.async_copy(
            ub_hbm.at[idx_u.at[c]], bias_u.at[c], sem))
    for cp in copies:
        cp.wait()

    pltpu.sync_copy(u_rows, rows_hbm.at[wid])
    pltpu.sync_copy(bias_u, ubias_hbm.at[wid])


@functools.partial(
    pl.kernel,
    out_type=jax.ShapeDtypeStruct((NW, NCHUNK, CHUNK), jnp.float32),
    mesh=_mesh,
    compiler_params=pltpu.CompilerParams(use_tc_tiling_on_sc=False),
    scratch_types=[
        pltpu.VMEM((NCHUNK, CHUNK), jnp.int32),
        pltpu.VMEM((BPW, K), jnp.float32),
        pltpu.VMEM((BPW, K), jnp.float32),
        pltpu.VMEM((NCHUNK, CHUNK), jnp.float32),
        pltpu.VMEM((NCHUNK, CHUNK), jnp.float32),
        pltpu.VMEM((NCHUNK, CHUNK), jnp.float32),
        pltpu.SemaphoreType.DMA,
    ],
)
def _mf_item(item_hbm, ie_hbm, ib_hbm, rows_hbm, ubias_hbm, out_hbm,
             idx_i, i_rows, u_rows, bias_u, bias_i, out_v, sem):
    wid = lax.axis_index("s") * NC + lax.axis_index("c")

    pltpu.sync_copy(item_hbm.at[wid], idx_i)
    copies = [
        pltpu.async_copy(rows_hbm.at[wid], u_rows, sem),
        pltpu.async_copy(ubias_hbm.at[wid], bias_u, sem),
    ]
    for c in range(NCHUNK):
        copies.append(pltpu.async_copy(
            ie_hbm.at[idx_i.at[c]], i_rows.at[pl.ds(c * CHUNK, CHUNK)], sem))
        copies.append(pltpu.async_copy(
            ib_hbm.at[idx_i.at[c]], bias_i.at[c], sem))
    for cp in copies:
        cp.wait()

    lane = lax.iota(jnp.int32, 16)

    def group_body(g, _):
        res = jnp.zeros((16,), jnp.float32)
        for j in range(16):
            e = g * 16 + j
            acc = jnp.zeros((16,), jnp.float32)
            for t in range(K // 16):
                acc = acc + (u_rows[e, pl.ds(t * 16, 16)]
                             * i_rows[e, pl.ds(t * 16, 16)])
            for sh in (1, 2, 4, 8):
                acc = acc + _permute(acc, lane ^ sh)
            res = jnp.where(lane == j, acc, res)
        c = g // (CHUNK // 16)
        sl = pl.ds((g % (CHUNK // 16)) * 16, 16)
        out_v[c, sl] = res + bias_u[c, sl] + bias_i[c, sl]
        return _

    lax.fori_loop(0, GROUPS, group_body, 0)

    pltpu.sync_copy(out_v, out_hbm.at[wid])


def kernel(user, item, user_embedding, item_embedding, user_bias, item_bias):
    user = user.astype(jnp.int32).reshape(NW, NCHUNK, CHUNK)
    item = item.astype(jnp.int32).reshape(NW, NCHUNK, CHUNK)
    ub = user_bias.reshape(-1)
    ib = item_bias.reshape(-1)
    ue_rm = _transpose_table(user_embedding.T)
    ie_rm = _transpose_table(item_embedding.T)
    u_rows, u_bias = _mf_user(user, ue_rm, ub)
    out = _mf_item(item, ie_rm, ib, u_rows, u_bias)
    return out.reshape(B)


# speedup vs baseline: 1.5957x; 1.5957x over previous
import functools

import jax
import jax.numpy as jnp
from jax import lax
from jax.experimental import pallas as pl
from jax.experimental.pallas import tpu as pltpu
from jax.experimental.pallas import tpu_sc as plsc

B = 16384
K = 64
NC = 2
NS = 16
NW = NC * NS
BPW = B // NW
CHUNK = 128
NCHUNK = BPW // CHUNK
GROUPS = BPW // 16

_mesh = plsc.VectorSubcoreMesh(core_axis_name="c", subcore_axis_name="s")

_GATHER_DNUMS = lax.GatherDimensionNumbers(
    offset_dims=(), collapsed_slice_dims=(0,), start_index_map=(0,))

TBLK = 4096


def _tp_body(x_ref, o_ref):
    i = lax.broadcasted_iota(jnp.int32, (K, K), 0)
    j = lax.broadcasted_iota(jnp.int32, (K, K), 1)
    eye = (i == j).astype(jnp.float32)
    o_ref[...] = lax.dot_general(
        x_ref[...], eye, (((0,), (0,)), ((), ())),
        preferred_element_type=jnp.float32,
        precision=lax.Precision.HIGHEST)


def _transpose_table(t_fm):
    nrows = t_fm.shape[1]
    grid = (nrows + TBLK - 1) // TBLK
    return pl.pallas_call(
        _tp_body,
        grid=(grid,),
        in_specs=[pl.BlockSpec((K, TBLK), lambda i: (0, i))],
        out_specs=pl.BlockSpec((TBLK, K), lambda i: (i, 0)),
        out_shape=jax.ShapeDtypeStruct((nrows, K), jnp.float32),
    )(t_fm)


def _permute(x, idx):
    return lax.gather(x, idx[:, None], _GATHER_DNUMS, (1,),
                      mode=lax.GatherScatterMode.PROMISE_IN_BOUNDS)


@functools.partial(
    pl.kernel,
    out_type=(jax.ShapeDtypeStruct((NW, BPW, K), jnp.float32),
              jax.ShapeDtypeStruct((NW, NCHUNK, CHUNK), jnp.float32)),
    mesh=_mesh,
    compiler_params=pltpu.CompilerParams(
        use_tc_tiling_on_sc=False, skip_device_barrier=True),
    scratch_types=[
        pltpu.VMEM((NCHUNK, CHUNK), jnp.int32),
        pltpu.VMEM((BPW, K), jnp.float32),
        pltpu.VMEM((NCHUNK, CHUNK), jnp.float32),
        pltpu.SemaphoreType.DMA,
    ],
)
def _mf_user(user_hbm, ue_hbm, ub_hbm, rows_hbm, ubias_hbm,
             idx_u, u_rows, bias_u, sem):
    wid = lax.axis_index("s") * NC + lax.axis_index("c")

    pltpu.sync_copy(user_hbm.at[wid], idx_u)
    copies = []
    for c in range(NCHUNK):
        copies.append(pltpu.async_copy(
            ue_hbm.at[idx_u.at[c]], u_rows.at[pl.ds(c * CHUNK, CHUNK)], sem))
        copies.append(pltpu.async_copy(
            ub_hbm.at[idx_u.at[c]], bias_u.at[c], sem))
    for cp in copies:
        cp.wait()

    pltpu.sync_copy(u_rows, rows_hbm.at[wid])
    pltpu.sync_copy(bias_u, ubias_hbm.at[wid])


@functools.partial(
    pl.kernel,
    out_type=jax.ShapeDtypeStruct((NW, NCHUNK, CHUNK), jnp.float32),
    mesh=_mesh,
    compiler_params=pltpu.CompilerParams(
        use_tc_tiling_on_sc=False, skip_device_barrier=True),
    scratch_types=[
        pltpu.VMEM((NCHUNK, CHUNK), jnp.int32),
        pltpu.VMEM((BPW, K), jnp.float32),
        pltpu.VMEM((BPW, K), jnp.float32),
        pltpu.VMEM((NCHUNK, CHUNK), jnp.float32),
        pltpu.VMEM((NCHUNK, CHUNK), jnp.float32),
        pltpu.VMEM((NCHUNK, CHUNK), jnp.float32),
        pltpu.SemaphoreType.DMA,
    ],
)
def _mf_item(item_hbm, ie_hbm, ib_hbm, rows_hbm, ubias_hbm, out_hbm,
             idx_i, i_rows, u_rows, bias_u, bias_i, out_v, sem):
    wid = lax.axis_index("s") * NC + lax.axis_index("c")

    pltpu.sync_copy(item_hbm.at[wid], idx_i)
    copies = [
        pltpu.async_copy(rows_hbm.at[wid], u_rows, sem),
        pltpu.async_copy(ubias_hbm.at[wid], bias_u, sem),
    ]
    for c in range(NCHUNK):
        copies.append(pltpu.async_copy(
            ie_hbm.at[idx_i.at[c]], i_rows.at[pl.ds(c * CHUNK, CHUNK)], sem))
        copies.append(pltpu.async_copy(
            ib_hbm.at[idx_i.at[c]], bias_i.at[c], sem))
    for cp in copies:
        cp.wait()

    lane = lax.iota(jnp.int32, 16)

    def group_body(g, _):
        res = jnp.zeros((16,), jnp.float32)
        for j in range(16):
            e = g * 16 + j
            acc = jnp.zeros((16,), jnp.float32)
            for t in range(K // 16):
                acc = acc + (u_rows[e, pl.ds(t * 16, 16)]
                             * i_rows[e, pl.ds(t * 16, 16)])
            for sh in (1, 2, 4, 8):
                acc = acc + _permute(acc, lane ^ sh)
            res = jnp.where(lane == j, acc, res)
        c = g // (CHUNK // 16)
        sl = pl.ds((g % (CHUNK // 16)) * 16, 16)
        out_v[c, sl] = res + bias_u[c, sl] + bias_i[c, sl]
        return _

    lax.fori_loop(0, GROUPS, group_body, 0)

    pltpu.sync_copy(out_v, out_hbm.at[wid])


def kernel(user, item, user_embedding, item_embedding, user_bias, item_bias):
    user = user.astype(jnp.int32).reshape(NW, NCHUNK, CHUNK)
    item = item.astype(jnp.int32).reshape(NW, NCHUNK, CHUNK)
    ub = user_bias.reshape(-1)
    ib = item_bias.reshape(-1)
    u_rows, u_bias = _mf_user(user, user_embedding, ub)
    out = _mf_item(item, item_embedding, ib, u_rows, u_bias)
    return out.reshape(B)
